# Initial kernel scaffold; baseline (speedup 1.0000x reference)
import jax
import jax.numpy as jnp
from jax.experimental import pallas as pl


def kernel(probas, labels):
    raise NotImplementedError("write your pallas kernel here")


# speedup vs baseline: 4.5238x; 4.5238x over previous
import functools

import jax
import jax.numpy as jnp
from jax.experimental import pallas as pl
from jax.experimental.pallas import tpu as pltpu

_B = 64
_N = 100000
_K = 11
_W = 4096
_NB = (_N + _W - 1) // _W
_S = _W // 128


def _topk_kernel(prob_ref, lab_ref, out_ref, state_ref, cacc_ref):
    i = pl.program_id(0)

    @pl.when(i == 0)
    def _init():
        state_ref[...] = jnp.full((_B, _K * 128), -jnp.inf, jnp.float32)
        cacc_ref[...] = jnp.zeros((_B, 128), jnp.float32)

    st = [state_ref[:, j * 128:(j + 1) * 128] for j in range(_K)]
    cacc = cacc_ref[...]
    labs = lab_ref[...]
    base = i * _W
    lane = jax.lax.broadcasted_iota(jnp.int32, (_B, 128), 1)
    for s in range(_S):
        x = prob_ref[:, s * 128:(s + 1) * 128]
        cols = base + s * 128 + lane
        valid = cols < _N
        cacc = cacc + jnp.where((cols == labs) & valid, x, 0.0)
        xm = jnp.where(valid, x, -jnp.inf)
        for j in range(_K):
            hi = jnp.maximum(st[j], xm)
            xm = jnp.minimum(st[j], xm)
            st[j] = hi
    for j in range(_K):
        state_ref[:, j * 128:(j + 1) * 128] = st[j]
    cacc_ref[...] = cacc

    @pl.when(i == _NB - 1)
    def _finish():
        a = state_ref[...]
        iota = jax.lax.broadcasted_iota(jnp.int32, (_B, _K * 128), 1)
        sum11 = jnp.zeros((_B, 1), jnp.float32)
        m = jnp.zeros((_B, 1), jnp.float32)
        for _ in range(_K):
            m = jnp.max(a, axis=1, keepdims=True)
            sum11 = sum11 + m
            pos = jnp.min(jnp.where(a == m, iota, _K * 128),
                          axis=1, keepdims=True)
            a = jnp.where(iota == pos, -jnp.inf, a)
        v11 = m
        c = jnp.sum(cacc_ref[...], axis=1, keepdims=True)
        sub = jnp.where(c >= v11, c, v11)
        loss_rows = (sum11 - sub) * 0.1 - c
        out_ref[0, 0] = jnp.sum(loss_rows) * (1.0 / _B)


@jax.jit
def kernel(probas, labels):
    lab2d = jnp.broadcast_to(labels.astype(jnp.int32)[:, None], (_B, 128))
    out = pl.pallas_call(
        _topk_kernel,
        grid=(_NB,),
        in_specs=[
            pl.BlockSpec((_B, _W), lambda i: (0, i)),
            pl.BlockSpec((_B, 128), lambda i: (0, 0)),
        ],
        out_specs=pl.BlockSpec(memory_space=pltpu.SMEM),
        out_shape=jax.ShapeDtypeStruct((1, 1), jnp.float32),
        scratch_shapes=[
            pltpu.VMEM((_B, _K * 128), jnp.float32),
            pltpu.VMEM((_B, 128), jnp.float32),
        ],
    )(probas, lab2d)
    return out[0, 0]
